# Initial kernel scaffold; baseline (speedup 1.0000x reference)
import jax
import jax.numpy as jnp
from jax.experimental import pallas as pl


def kernel(x, codebook):
    raise NotImplementedError("write your pallas kernel here")


# speedup vs baseline: 1.2885x; 1.2885x over previous
import functools

import jax
import jax.numpy as jnp
from jax import lax
from jax.experimental import pallas as pl
from jax.experimental.pallas import tpu as pltpu
from jax.experimental.pallas import tpu_sc as plsc

_NB = 8192
_CD = 256
_TM = 256


def _dist_argmin_body(n_total, x_ref, cb_ref, cn_ref, idx_ref, loss_ref,
                      acc_ref):
    i = pl.program_id(0)
    xt = x_ref[...]
    mm = lax.dot_general(xt, cb_ref[...], (((1,), (1,)), ((), ())),
                         preferred_element_type=jnp.float32)
    xn = jnp.sum(xt * xt, axis=1, keepdims=True)
    d = xn - 2.0 * mm + cn_ref[...]
    m = jnp.min(d, axis=1, keepdims=True)
    col = lax.broadcasted_iota(jnp.int32, d.shape, 1)
    idx_ref[...] = jnp.min(jnp.where(d == m, col, _NB), axis=1)

    @pl.when(i == 0)
    def _init():
        acc_ref[0] = 0.0

    acc_ref[0] += jnp.sum(m)

    @pl.when(i == pl.num_programs(0) - 1)
    def _fin():
        loss_ref[0] = acc_ref[0] / float(n_total * _CD)


def _argmin_call(x_flat, codebook, cnorm):
    n = x_flat.shape[0]
    assert n % _TM == 0
    return pl.pallas_call(
        functools.partial(_dist_argmin_body, n),
        grid=(n // _TM,),
        in_specs=[
            pl.BlockSpec((_TM, _CD), lambda i: (i, 0)),
            pl.BlockSpec((_NB, _CD), lambda i: (0, 0)),
            pl.BlockSpec((1, _NB), lambda i: (0, 0)),
        ],
        out_specs=[
            pl.BlockSpec((_TM,), lambda i: (i,)),
            pl.BlockSpec(memory_space=pltpu.SMEM),
        ],
        out_shape=[
            jax.ShapeDtypeStruct((n,), jnp.int32),
            jax.ShapeDtypeStruct((1,), jnp.float32),
        ],
        scratch_shapes=[pltpu.SMEM((1,), jnp.float32)],
    )(x_flat, codebook, cnorm)


_NC = 2
_NS = 16
_NW = _NC * _NS
_CHUNK = 88


def _sc_gather_body(bpw, cb_hbm, idx_hbm, out_hbm, idx_v, rows_v, sem):
    wid = lax.axis_index("s") * _NC + lax.axis_index("c")
    base = wid * bpw
    pltpu.sync_copy(idx_hbm.at[pl.ds(base, bpw)], idx_v)
    copies = []
    for j in range(bpw // _CHUNK):
        copies.append(pltpu.async_copy(
            cb_hbm.at[idx_v.at[pl.ds(j * _CHUNK, _CHUNK)]],
            rows_v.at[pl.ds(j * _CHUNK, _CHUNK)],
            sem))
    for c in copies:
        c.wait()
    pltpu.sync_copy(rows_v, out_hbm.at[pl.ds(base, bpw)])


def _gather_call(codebook, idx):
    n = idx.shape[0]
    assert n % (_NW * _CHUNK) == 0
    bpw = n // _NW
    mesh = plsc.VectorSubcoreMesh(core_axis_name="c", subcore_axis_name="s")
    f = functools.partial(
        pl.kernel,
        mesh=mesh,
        out_type=jax.ShapeDtypeStruct((n, _CD), jnp.float32),
        scratch_types=[
            pltpu.VMEM((bpw,), jnp.int32),
            pltpu.VMEM((bpw, _CD), jnp.float32),
            pltpu.SemaphoreType.DMA,
        ],
    )(functools.partial(_sc_gather_body, bpw))
    return f(codebook, idx)


def kernel(x, codebook):
    B, C, J, T = x.shape
    x_flat = jnp.transpose(x, (0, 2, 3, 1)).reshape(-1, C)
    cnorm = jnp.sum(codebook ** 2, axis=-1)[None, :]
    idx, loss = _argmin_call(x_flat, codebook, cnorm)
    q_flat = _gather_call(codebook, idx)
    x_quantized = jnp.transpose(q_flat.reshape(B, J, T, C), (0, 3, 1, 2))
    return x_quantized, loss[0]
